# Initial kernel scaffold; baseline (speedup 1.0000x reference)
import jax
import jax.numpy as jnp
from jax.experimental import pallas as pl


def kernel(input_ids, embedding_weight, subword_weight):
    raise NotImplementedError("write your pallas kernel here")


# speedup vs baseline: 3.7707x; 3.7707x over previous
import functools

import jax
import jax.numpy as jnp
from jax import lax
from jax.experimental import pallas as pl
from jax.experimental.pallas import tpu as pltpu
from jax.experimental.pallas import tpu_sc as plsc

_VOCAB = 100000
_D = 64
_N = 4096 * 50

_NC = 2
_NS = 16
_NW = _NC * _NS
_PER_W = _N // _NW
_CHUNK = 128
_NCHUNK = _PER_W // _CHUNK

_mesh = plsc.VectorSubcoreMesh(core_axis_name="c", subcore_axis_name="s")


@functools.partial(
    pl.kernel,
    mesh=_mesh,
    out_type=jax.ShapeDtypeStruct((_N, _D), jnp.float32),
    compiler_params=pltpu.CompilerParams(use_tc_tiling_on_sc=False),
    scratch_types=[
        pltpu.VMEM((_CHUNK,), jnp.int32),
        pltpu.VMEM((_CHUNK, _D), jnp.float32),
        pltpu.SemaphoreType.DMA,
    ],
)
def _sc_gather(idx_hbm, table_hbm, out_hbm, idx_v, rows_v, sem):
    wid = lax.axis_index("s") * _NC + lax.axis_index("c")
    base = wid * _PER_W

    def body(i, carry):
        off = base + i * _CHUNK
        pltpu.sync_copy(idx_hbm.at[pl.ds(off, _CHUNK)], idx_v)
        pltpu.async_copy(table_hbm.at[idx_v], rows_v, sem).wait()
        pltpu.sync_copy(rows_v, out_hbm.at[pl.ds(off, _CHUNK)])
        return carry

    lax.fori_loop(0, _NCHUNK, body, 0)


def kernel(input_ids, embedding_weight, subword_weight):
    flat = input_ids.reshape(-1).astype(jnp.int32)
    out = _sc_gather(flat, embedding_weight)
    return out.reshape(input_ids.shape[0], input_ids.shape[1], _D)


# speedup vs baseline: 5.1200x; 1.3578x over previous
import functools

import jax
import jax.numpy as jnp
from jax import lax
from jax.experimental import pallas as pl
from jax.experimental.pallas import tpu as pltpu
from jax.experimental.pallas import tpu_sc as plsc

_B = 4096
_S = 50
_D = 64
_DP = 128
_L = 16

_NC = 2
_NS = 16
_NW = _NC * _NS
_BPW = _B // _NW
_NBUF = 4
_NROUND = _BPW // _NBUF

_mesh = plsc.VectorSubcoreMesh(core_axis_name="c", subcore_axis_name="s")


@functools.partial(
    pl.kernel,
    mesh=_mesh,
    out_type=jax.ShapeDtypeStruct((_B, _S, _D), jnp.float32),
    compiler_params=pltpu.CompilerParams(use_tc_tiling_on_sc=True),
    scratch_types=[
        pltpu.VMEM((_BPW, _S), jnp.int32),
        pltpu.VMEM((_NBUF, _S, _DP), jnp.float32),
        pltpu.VMEM((_NBUF, _S, _D), jnp.float32),
        pltpu.SemaphoreType.DMA((_NBUF,)),
        pltpu.SemaphoreType.DMA((_NBUF,)),
    ],
)
def _sc_gather(ids_hbm, tab_hbm, out_hbm, idx_v, rows_v, obuf, gsem, osem):
    wid = lax.axis_index("s") * _NC + lax.axis_index("c")
    b0 = wid * _BPW

    pltpu.sync_copy(ids_hbm.at[pl.ds(b0, _BPW)], idx_v)

    for r in range(_NBUF):
        pltpu.async_copy(tab_hbm.at[idx_v.at[r]], rows_v.at[r], gsem.at[r])

    def round_(rr, carry):
        j0 = rr * _NBUF
        for r in range(_NBUF):
            j = j0 + r
            pltpu.make_async_copy(
                tab_hbm.at[idx_v.at[j]], rows_v.at[r], gsem.at[r]
            ).wait()

            @pl.when(rr > 0)
            def _():
                pltpu.make_async_copy(
                    obuf.at[r], out_hbm.at[b0 + j - _NBUF], osem.at[r]
                ).wait()

            def compact(s, c):
                for k in range(_D // _L):
                    obuf[r, s, pl.ds(k * _L, _L)] = rows_v[r, s, pl.ds(k * _L, _L)]
                return c

            lax.fori_loop(0, _S, compact, 0)

            @pl.when(rr < _NROUND - 1)
            def _():
                pltpu.async_copy(
                    tab_hbm.at[idx_v.at[j + _NBUF]], rows_v.at[r], gsem.at[r]
                )

            pltpu.async_copy(obuf.at[r], out_hbm.at[b0 + j], osem.at[r])
        return carry

    lax.fori_loop(0, _NROUND, round_, 0)

    for r in range(_NBUF):
        pltpu.make_async_copy(
            obuf.at[r], out_hbm.at[b0 + _BPW - _NBUF + r], osem.at[r]
        ).wait()


def kernel(input_ids, embedding_weight, subword_weight):
    tab = jnp.pad(embedding_weight, ((0, 0), (0, _DP - _D)))
    return _sc_gather(input_ids.astype(jnp.int32), tab)


# speedup vs baseline: 7.2168x; 1.4095x over previous
import functools

import jax
import jax.numpy as jnp
from jax import lax
from jax.experimental import pallas as pl
from jax.experimental.pallas import tpu as pltpu
from jax.experimental.pallas import tpu_sc as plsc

_B = 4096
_S = 50
_D = 64
_V = 100000
_L = 16

_NC = 2
_NS = 16
_NW = _NC * _NS
_HALF = _B // 2
_DPW = _D // (_NW // 2)
_NBUF = 5
_NROUND = _S // _NBUF
_NG = _HALF // _L

_mesh = plsc.VectorSubcoreMesh(core_axis_name="c", subcore_axis_name="s")


@functools.partial(
    pl.kernel,
    mesh=_mesh,
    out_type=jax.ShapeDtypeStruct((_S, _D, _B), jnp.float32),
    compiler_params=pltpu.CompilerParams(
        use_tc_tiling_on_sc=True, needs_layout_passes=False
    ),
    scratch_types=[
        pltpu.VMEM((1, _V), jnp.float32),
        pltpu.VMEM((_NBUF, 1, _HALF), jnp.int32),
        pltpu.VMEM((_NBUF, 1, _HALF), jnp.float32),
        pltpu.SemaphoreType.DMA((_NBUF,)),
        pltpu.SemaphoreType.DMA((_NBUF,)),
    ],
)
def _sc_lookup(ids_hbm, tab_hbm, out_hbm, row_v, idx_v, obuf, isem, osem):
    wid = lax.axis_index("s") * _NC + lax.axis_index("c")
    half = wid // (_NW // 2)
    b0 = half * _HALF
    d0 = (wid % (_NW // 2)) * _DPW

    for r in range(_NBUF):
        pltpu.async_copy(
            ids_hbm.at[pl.ds(r, 1), pl.ds(b0, _HALF)], idx_v.at[r], isem.at[r]
        )

    for dd in range(_DPW):
        d = d0 + dd
        pltpu.sync_copy(tab_hbm.at[pl.ds(d, 1)], row_v)

        def round_(rr, carry):
            s0 = rr * _NBUF
            for r in range(_NBUF):
                s = s0 + r
                pltpu.make_async_copy(
                    ids_hbm.at[pl.ds(s, 1), pl.ds(b0, _HALF)], idx_v.at[r], isem.at[r]
                ).wait()

                @pl.when(jnp.logical_or(rr > 0, dd > 0))
                def _():
                    pltpu.make_async_copy(
                        obuf.at[r], out_hbm.at[s, pl.ds(d, 1), pl.ds(b0, _HALF)], osem.at[r]
                    ).wait()

                def gather(g, c):
                    for u in range(8):
                        o = (g * 8 + u) * _L
                        idx16 = idx_v[r, 0, pl.ds(o, _L)]
                        obuf[r, 0, pl.ds(o, _L)] = plsc.load_gather(
                            row_v, [idx16 * 0, idx16]
                        )
                    return c

                lax.fori_loop(0, _NG // 8, gather, 0)

                pltpu.async_copy(
                    obuf.at[r], out_hbm.at[s, pl.ds(d, 1), pl.ds(b0, _HALF)], osem.at[r]
                )

                s_next = s + _NBUF
                nxt = jnp.where(s_next < _S, s_next, s_next - _S)

                @pl.when(jnp.logical_or(rr < _NROUND - 1, dd < _DPW - 1))
                def _():
                    pltpu.async_copy(
                        ids_hbm.at[pl.ds(nxt, 1), pl.ds(b0, _HALF)], idx_v.at[r], isem.at[r]
                    )

            return carry

        lax.fori_loop(0, _NROUND, round_, 0)

    for r in range(_NBUF):
        pltpu.make_async_copy(
            obuf.at[r],
            out_hbm.at[_S - _NBUF + r, pl.ds(d0 + _DPW - 1, 1), pl.ds(b0, _HALF)],
            osem.at[r],
        ).wait()


def kernel(input_ids, embedding_weight, subword_weight):
    ids_t = input_ids.T.astype(jnp.int32)
    tab_t = embedding_weight.T
    out_t = _sc_lookup(ids_t, tab_t)
    return out_t.transpose(2, 0, 1)
